# baseline (speedup 1.0000x reference)
import functools

import jax
import jax.numpy as jnp
from jax import lax
from jax.experimental import pallas as pl
from jax.experimental.pallas import tpu as pltpu
from jax.experimental.pallas import tpu_sc as plsc

DICT_NUM = 10
DICT_DIM = 80
COMMON = 96

_NC = 2
_NS = 16
_NW = _NC * _NS
_CHUNK = 64


def _sc_gather(emb, idx):
    vocab, d = emb.shape
    b = idx.shape[0]
    b_per_w = b // _NW
    mesh = plsc.VectorSubcoreMesh(core_axis_name="c", subcore_axis_name="s")

    @functools.partial(
        pl.kernel,
        mesh=mesh,
        out_type=jax.ShapeDtypeStruct((b, d), jnp.float32),
        scratch_types=[
            pltpu.VMEM((b_per_w,), jnp.int32),
            pltpu.SemaphoreType.DMA,
        ],
    )
    def gather_kernel(table_hbm, idx_hbm, out_hbm, idx_v, sem):
        wid = lax.axis_index("s") * _NC + lax.axis_index("c")
        base = wid * b_per_w
        pltpu.sync_copy(idx_hbm.at[pl.ds(base, b_per_w)], idx_v)

        @pl.loop(0, b_per_w, step=16)
        def _(c):
            vec = idx_v[pl.ds(c, 16)]
            for j in range(16):
                s = lax.squeeze(lax.slice(vec, (j,), (j + 1,)), (0,))
                pltpu.make_async_copy(
                    table_hbm.at[s], out_hbm.at[base + c + j], sem
                ).start()
            for j in range(16):
                pltpu.make_async_copy(
                    table_hbm.at[0], out_hbm.at[base + c + j], sem
                ).wait()

    return gather_kernel(emb, idx)


def _lrelu(x):
    return jnp.where(x >= 0, x, 0.01 * x)


def _pad_body(src_ref, dst_ref):
    dst_ref[:, :800] = src_ref[...]
    dst_ref[:, 800:] = jnp.zeros((src_ref.shape[0], 96), jnp.float32)


def _pad_table(emb):
    vocab = emb.shape[0]
    rb = 1000
    return pl.pallas_call(
        _pad_body,
        grid=(vocab // rb,),
        in_specs=[pl.BlockSpec((rb, 800), lambda i: (i, 0))],
        out_specs=pl.BlockSpec((rb, 896), lambda i: (i, 0)),
        out_shape=jax.ShapeDtypeStruct((vocab, 896), jnp.float32),
    )(emb)


def _dense_body(fc_ref, fl_ref, ve_ref, w1_ref, b1_ref, w2_ref, b2_ref,
                w3_ref, b3_ref, wloc_ref, bloc_ref, wcls_ref,
                cls_ref, cos_ref, vcomp_ref, vlc_ref, vlcm_ref):
    f32 = jnp.float32
    x = fc_ref[...]
    h = _lrelu(jnp.dot(x, w1_ref[...], preferred_element_type=f32) + b1_ref[...])
    h = _lrelu(jnp.dot(h, w2_ref[...], preferred_element_type=f32) + b2_ref[...])
    v_comp = _lrelu(jnp.dot(h, w3_ref[...], preferred_element_type=f32) + b3_ref[...])
    v_loc = _lrelu(jnp.dot(fl_ref[...], wloc_ref[...], preferred_element_type=f32)
                   + bloc_ref[...])

    ve = ve_ref[...]
    n1 = jnp.sqrt(jnp.sum(v_comp * v_comp, axis=1, keepdims=True))

    best = None
    vlcm = None
    pieces = []
    for k in range(DICT_NUM):
        ve_k = ve[:, k * DICT_DIM:(k + 1) * DICT_DIM]
        cat_k = jnp.concatenate([v_loc, ve_k], axis=1)
        pieces.append(cat_k)
        dot_k = jnp.sum(v_comp * cat_k, axis=1, keepdims=True)
        n2_k = jnp.sqrt(jnp.sum(cat_k * cat_k, axis=1, keepdims=True))
        cos_k = dot_k / jnp.maximum(n1 * n2_k, 1e-8)
        if best is None:
            best = cos_k
            vlcm = cat_k
        else:
            upd = cos_k > best
            best = jnp.where(upd, cos_k, best)
            vlcm = jnp.where(upd, cat_k, vlcm)

    vlc_ref[...] = jnp.concatenate(pieces, axis=1)
    cos_ref[...] = best
    vcomp_ref[...] = v_comp
    vlcm_ref[...] = vlcm
    v_diff = jnp.abs(vlcm - v_comp)
    cls_ref[...] = jnp.dot(v_diff, wcls_ref[...], preferred_element_type=f32)


def _dense(feat_comp, feat_loc, v_emb, W1, b1, W2, b2, W3, b3, Wloc, bloc, Wcls):
    b = feat_comp.shape[0]
    bb = 1024
    grid = (b // bb,)
    f32 = jnp.float32

    def row_spec(cols):
        return pl.BlockSpec((bb, cols), lambda i: (i, 0))

    def full_spec(shape):
        return pl.BlockSpec(shape, lambda i: (0, 0))

    weights = [W1.T, b1.reshape(1, -1), W2.T, b2.reshape(1, -1),
               W3.T, b3.reshape(1, -1), Wloc.T, bloc.reshape(1, -1), Wcls.T]

    out = pl.pallas_call(
        _dense_body,
        grid=grid,
        in_specs=[
            row_spec(feat_comp.shape[1]),
            row_spec(feat_loc.shape[1]),
            row_spec(v_emb.shape[1]),
        ] + [full_spec(w.shape) for w in weights],
        out_specs=[
            row_spec(2),
            row_spec(1),
            row_spec(COMMON),
            row_spec(DICT_NUM * COMMON),
            row_spec(COMMON),
        ],
        out_shape=[
            jax.ShapeDtypeStruct((b, 2), f32),
            jax.ShapeDtypeStruct((b, 1), f32),
            jax.ShapeDtypeStruct((b, COMMON), f32),
            jax.ShapeDtypeStruct((b, DICT_NUM * COMMON), f32),
            jax.ShapeDtypeStruct((b, COMMON), f32),
        ],
    )(feat_comp, feat_loc, v_emb, *weights)
    return out


@jax.jit
def kernel(feat_comp, feat_loc, id_loc, W1, b1, W2, b2, W3, b3, emb, Wloc, bloc, Wcls):
    v_emb = _sc_gather(emb, id_loc.astype(jnp.int32))
    cls, cos, vcomp, vlc, vlcm = _dense(
        feat_comp, feat_loc, v_emb, W1, b1, W2, b2, W3, b3, Wloc, bloc, Wcls)
    b = feat_comp.shape[0]
    return (cls, cos, vcomp, vlc.reshape(b, DICT_NUM, COMMON), vlcm)


# speedup vs baseline: 2.6271x; 2.6271x over previous
import functools

import jax
import jax.numpy as jnp
from jax import lax
from jax.experimental import pallas as pl
from jax.experimental.pallas import tpu as pltpu
from jax.experimental.pallas import tpu_sc as plsc

DICT_NUM = 10
DICT_DIM = 80
COMMON = 96

_NC = 2
_NS = 16
_NW = _NC * _NS
_CHUNK = 64


def _sc_gather(emb, idx):
    vocab, d = emb.shape
    b = idx.shape[0]
    b_per_w = b // _NW
    mesh = plsc.VectorSubcoreMesh(core_axis_name="c", subcore_axis_name="s")

    @functools.partial(
        pl.kernel,
        mesh=mesh,
        out_type=jax.ShapeDtypeStruct((b, d), jnp.float32),
        scratch_types=[
            pltpu.VMEM((b_per_w,), jnp.int32),
            pltpu.VMEM((_CHUNK, d), jnp.float32),
            pltpu.SemaphoreType.DMA,
        ],
    )
    def gather_kernel(table_hbm, idx_hbm, out_hbm, idx_v, rows_v, sem):
        wid = lax.axis_index("s") * _NC + lax.axis_index("c")
        base = wid * b_per_w
        pltpu.sync_copy(idx_hbm.at[pl.ds(base, b_per_w)], idx_v)

        @pl.loop(0, b_per_w, step=_CHUNK)
        def _(c):
            pltpu.async_copy(
                table_hbm.at[idx_v.at[pl.ds(c, _CHUNK)]], rows_v, sem
            ).wait()
            pltpu.sync_copy(rows_v, out_hbm.at[pl.ds(base + c, _CHUNK)])

    return gather_kernel(emb, idx)


def _lrelu(x):
    return jnp.where(x >= 0, x, 0.01 * x)


def _pad_body(src_ref, dst_ref):
    dst_ref[:, :800] = src_ref[...]
    dst_ref[:, 800:] = jnp.zeros((src_ref.shape[0], 96), jnp.float32)


def _pad_table(emb):
    vocab = emb.shape[0]
    rb = 1000
    return pl.pallas_call(
        _pad_body,
        grid=(vocab // rb,),
        in_specs=[pl.BlockSpec((rb, 800), lambda i: (i, 0))],
        out_specs=pl.BlockSpec((rb, 896), lambda i: (i, 0)),
        out_shape=jax.ShapeDtypeStruct((vocab, 896), jnp.float32),
    )(emb)


def _dense_body(fc_ref, fl_ref, ve_ref, w1_ref, b1_ref, w2_ref, b2_ref,
                w3_ref, b3_ref, wloc_ref, bloc_ref, wcls_ref,
                cls_ref, cos_ref, vcomp_ref, vlc_ref, vlcm_ref):
    f32 = jnp.float32
    x = fc_ref[...]
    h = _lrelu(jnp.dot(x, w1_ref[...], preferred_element_type=f32) + b1_ref[...])
    h = _lrelu(jnp.dot(h, w2_ref[...], preferred_element_type=f32) + b2_ref[...])
    v_comp = _lrelu(jnp.dot(h, w3_ref[...], preferred_element_type=f32) + b3_ref[...])
    v_loc = _lrelu(jnp.dot(fl_ref[...], wloc_ref[...], preferred_element_type=f32)
                   + bloc_ref[...])

    ve = ve_ref[...]
    n1 = jnp.sqrt(jnp.sum(v_comp * v_comp, axis=1, keepdims=True))

    best = None
    vlcm = None
    for k in range(DICT_NUM):
        ve_k = ve[:, k * DICT_DIM:(k + 1) * DICT_DIM]
        cat_k = jnp.concatenate([v_loc, ve_k], axis=1)
        vlc_ref[:, k, :] = cat_k
        dot_k = jnp.sum(v_comp * cat_k, axis=1, keepdims=True)
        n2_k = jnp.sqrt(jnp.sum(cat_k * cat_k, axis=1, keepdims=True))
        cos_k = dot_k / jnp.maximum(n1 * n2_k, 1e-8)
        if best is None:
            best = cos_k
            vlcm = cat_k
        else:
            upd = cos_k > best
            best = jnp.where(upd, cos_k, best)
            vlcm = jnp.where(upd, cat_k, vlcm)

    cos_ref[...] = best
    vcomp_ref[...] = v_comp
    vlcm_ref[...] = vlcm
    v_diff = jnp.abs(vlcm - v_comp)
    cls_ref[...] = jnp.dot(v_diff, wcls_ref[...], preferred_element_type=f32)


def _dense(feat_comp, feat_loc, v_emb, W1, b1, W2, b2, W3, b3, Wloc, bloc, Wcls):
    b = feat_comp.shape[0]
    bb = 1024
    grid = (b // bb,)
    f32 = jnp.float32

    def row_spec(cols):
        return pl.BlockSpec((bb, cols), lambda i: (i, 0))

    def full_spec(shape):
        return pl.BlockSpec(shape, lambda i: (0, 0))

    weights = [W1.T, b1.reshape(1, -1), W2.T, b2.reshape(1, -1),
               W3.T, b3.reshape(1, -1), Wloc.T, bloc.reshape(1, -1), Wcls.T]

    out = pl.pallas_call(
        _dense_body,
        grid=grid,
        in_specs=[
            row_spec(feat_comp.shape[1]),
            row_spec(feat_loc.shape[1]),
            row_spec(v_emb.shape[1]),
        ] + [full_spec(w.shape) for w in weights],
        out_specs=[
            row_spec(2),
            row_spec(1),
            row_spec(COMMON),
            pl.BlockSpec((bb, DICT_NUM, COMMON), lambda i: (i, 0, 0)),
            row_spec(COMMON),
        ],
        out_shape=[
            jax.ShapeDtypeStruct((b, 2), f32),
            jax.ShapeDtypeStruct((b, 1), f32),
            jax.ShapeDtypeStruct((b, COMMON), f32),
            jax.ShapeDtypeStruct((b, DICT_NUM, COMMON), f32),
            jax.ShapeDtypeStruct((b, COMMON), f32),
        ],
    )(feat_comp, feat_loc, v_emb, *weights)
    return out


@jax.jit
def kernel(feat_comp, feat_loc, id_loc, W1, b1, W2, b2, W3, b3, emb, Wloc, bloc, Wcls):
    emb_p = _pad_table(emb)
    v_emb = _sc_gather(emb_p, id_loc.astype(jnp.int32))
    cls, cos, vcomp, vlc, vlcm = _dense(
        feat_comp, feat_loc, v_emb, W1, b1, W2, b2, W3, b3, Wloc, bloc, Wcls)
    return (cls, cos, vcomp, vlc, vlcm)


# speedup vs baseline: 4.0619x; 1.5461x over previous
import functools

import jax
import jax.numpy as jnp
from jax import lax
from jax.experimental import pallas as pl
from jax.experimental.pallas import tpu as pltpu
from jax.experimental.pallas import tpu_sc as plsc

DICT_NUM = 10
DICT_DIM = 80
COMMON = 96

_NC = 2
_NS = 16
_NW = _NC * _NS
_CHUNK = 64


def _sc_gather(emb, idx):
    vocab, d = emb.shape
    b = idx.shape[0]
    b_per_w = b // _NW
    mesh = plsc.VectorSubcoreMesh(core_axis_name="c", subcore_axis_name="s")

    @functools.partial(
        pl.kernel,
        mesh=mesh,
        out_type=jax.ShapeDtypeStruct((b, d), jnp.float32),
        scratch_types=[
            pltpu.VMEM((b_per_w,), jnp.int32),
            pltpu.VMEM((_CHUNK, d), jnp.float32),
            pltpu.SemaphoreType.DMA,
        ],
    )
    def gather_kernel(table_hbm, idx_hbm, out_hbm, idx_v, rows_v, sem):
        wid = lax.axis_index("s") * _NC + lax.axis_index("c")
        base = wid * b_per_w
        pltpu.sync_copy(idx_hbm.at[pl.ds(base, b_per_w)], idx_v)

        @pl.loop(0, b_per_w, step=_CHUNK)
        def _(c):
            pltpu.async_copy(
                table_hbm.at[idx_v.at[pl.ds(c, _CHUNK)]], rows_v, sem
            ).wait()
            pltpu.sync_copy(rows_v, out_hbm.at[pl.ds(base + c, _CHUNK)])

    return gather_kernel(emb, idx)


def _lrelu(x):
    return jnp.where(x >= 0, x, 0.01 * x)


def _pad_body(src_ref, dst_ref):
    dst_ref[:, :800] = src_ref[...].T
    dst_ref[:, 800:] = jnp.zeros((dst_ref.shape[0], 96), jnp.float32)


def _pad_table(emb):
    vocab = emb.shape[0]
    rb = 1024
    embt = emb.T
    return pl.pallas_call(
        _pad_body,
        grid=(pl.cdiv(vocab, rb),),
        in_specs=[pl.BlockSpec((800, rb), lambda i: (0, i))],
        out_specs=pl.BlockSpec((rb, 896), lambda i: (i, 0)),
        out_shape=jax.ShapeDtypeStruct((vocab, 896), jnp.float32),
    )(embt)


def _dense_body(fc_ref, fl_ref, ve_ref, w1_ref, b1_ref, w2_ref, b2_ref,
                w3_ref, b3_ref, wloc_ref, bloc_ref, wcls_ref,
                cls_ref, cos_ref, vcomp_ref, vlc_ref, vlcm_ref):
    f32 = jnp.float32
    x = fc_ref[...]
    h = _lrelu(jnp.dot(x, w1_ref[...], preferred_element_type=f32) + b1_ref[...])
    h = _lrelu(jnp.dot(h, w2_ref[...], preferred_element_type=f32) + b2_ref[...])
    v_comp = _lrelu(jnp.dot(h, w3_ref[...], preferred_element_type=f32) + b3_ref[...])
    v_loc = _lrelu(jnp.dot(fl_ref[...], wloc_ref[...], preferred_element_type=f32)
                   + bloc_ref[...])

    ve = ve_ref[...]
    n1 = jnp.sqrt(jnp.sum(v_comp * v_comp, axis=1, keepdims=True))

    best = None
    vlcm = None
    for k in range(DICT_NUM):
        ve_k = ve[:, k * DICT_DIM:(k + 1) * DICT_DIM]
        cat_k = jnp.concatenate([v_loc, ve_k], axis=1)
        vlc_ref[:, k, :] = cat_k
        dot_k = jnp.sum(v_comp * cat_k, axis=1, keepdims=True)
        n2_k = jnp.sqrt(jnp.sum(cat_k * cat_k, axis=1, keepdims=True))
        cos_k = dot_k / jnp.maximum(n1 * n2_k, 1e-8)
        if best is None:
            best = cos_k
            vlcm = cat_k
        else:
            upd = cos_k > best
            best = jnp.where(upd, cos_k, best)
            vlcm = jnp.where(upd, cat_k, vlcm)

    cos_ref[...] = best
    vcomp_ref[...] = v_comp
    vlcm_ref[...] = vlcm
    v_diff = jnp.abs(vlcm - v_comp)
    cls_ref[...] = jnp.dot(v_diff, wcls_ref[...], preferred_element_type=f32)


def _dense(feat_comp, feat_loc, v_emb, W1, b1, W2, b2, W3, b3, Wloc, bloc, Wcls):
    b = feat_comp.shape[0]
    bb = 1024
    grid = (b // bb,)
    f32 = jnp.float32

    def row_spec(cols):
        return pl.BlockSpec((bb, cols), lambda i: (i, 0))

    def full_spec(shape):
        return pl.BlockSpec(shape, lambda i: (0, 0))

    weights = [W1.T, b1.reshape(1, -1), W2.T, b2.reshape(1, -1),
               W3.T, b3.reshape(1, -1), Wloc.T, bloc.reshape(1, -1), Wcls.T]

    out = pl.pallas_call(
        _dense_body,
        grid=grid,
        in_specs=[
            row_spec(feat_comp.shape[1]),
            row_spec(feat_loc.shape[1]),
            row_spec(v_emb.shape[1]),
        ] + [full_spec(w.shape) for w in weights],
        out_specs=[
            row_spec(2),
            row_spec(1),
            row_spec(COMMON),
            pl.BlockSpec((bb, DICT_NUM, COMMON), lambda i: (i, 0, 0)),
            row_spec(COMMON),
        ],
        out_shape=[
            jax.ShapeDtypeStruct((b, 2), f32),
            jax.ShapeDtypeStruct((b, 1), f32),
            jax.ShapeDtypeStruct((b, COMMON), f32),
            jax.ShapeDtypeStruct((b, DICT_NUM, COMMON), f32),
            jax.ShapeDtypeStruct((b, COMMON), f32),
        ],
    )(feat_comp, feat_loc, v_emb, *weights)
    return out


@jax.jit
def kernel(feat_comp, feat_loc, id_loc, W1, b1, W2, b2, W3, b3, emb, Wloc, bloc, Wcls):
    emb_p = _pad_table(emb)
    v_emb = _sc_gather(emb_p, id_loc.astype(jnp.int32))
    cls, cos, vcomp, vlc, vlcm = _dense(
        feat_comp, feat_loc, v_emb, W1, b1, W2, b2, W3, b3, Wloc, bloc, Wcls)
    return (cls, cos, vcomp, vlc, vlcm)


# speedup vs baseline: 6.7887x; 1.6713x over previous
import functools

import jax
import jax.numpy as jnp
from jax import lax
from jax.experimental import pallas as pl
from jax.experimental.pallas import tpu as pltpu
from jax.experimental.pallas import tpu_sc as plsc

DICT_NUM = 10
DICT_DIM = 80
COMMON = 96

_NC = 2
_NS = 16
_NW = _NC * _NS
_CHUNK = 64


def _sc_gather(emb, idx):
    vocab, d = emb.shape
    b = idx.shape[0]
    b_per_w = b // _NW
    mesh = plsc.VectorSubcoreMesh(core_axis_name="c", subcore_axis_name="s")

    @functools.partial(
        pl.kernel,
        mesh=mesh,
        out_type=jax.ShapeDtypeStruct((b, d), jnp.float32),
        scratch_types=[
            pltpu.VMEM((b_per_w,), jnp.int32),
            pltpu.VMEM((_CHUNK, d), jnp.float32),
            pltpu.SemaphoreType.DMA,
        ],
    )
    def gather_kernel(table_hbm, idx_hbm, out_hbm, idx_v, rows_v, sem):
        wid = lax.axis_index("s") * _NC + lax.axis_index("c")
        base = wid * b_per_w
        pltpu.sync_copy(idx_hbm.at[pl.ds(base, b_per_w)], idx_v)

        @pl.loop(0, b_per_w, step=_CHUNK)
        def _(c):
            pltpu.async_copy(
                table_hbm.at[idx_v.at[pl.ds(c, _CHUNK)]], rows_v, sem
            ).wait()
            pltpu.sync_copy(rows_v, out_hbm.at[pl.ds(base + c, _CHUNK)])

    return gather_kernel(emb, idx)


def _lrelu(x):
    return jnp.where(x >= 0, x, 0.01 * x)


def _pad_body(src_ref, dst_ref):
    dst_ref[:, :800] = src_ref[...].T
    dst_ref[:, 800:] = jnp.zeros((dst_ref.shape[0], 96), jnp.float32)


def _pad_table(emb):
    vocab = emb.shape[0]
    rb = 1024
    embt = emb.T
    return pl.pallas_call(
        _pad_body,
        grid=(pl.cdiv(vocab, rb),),
        in_specs=[pl.BlockSpec((800, rb), lambda i: (0, i))],
        out_specs=pl.BlockSpec((rb, 896), lambda i: (i, 0)),
        out_shape=jax.ShapeDtypeStruct((vocab, 896), jnp.float32),
    )(embt)


def _dense_body(fct_ref, flt_ref, ve_ref, w1_ref, b1_ref, w2_ref, b2_ref,
                w3_ref, b3_ref, wloc_ref, bloc_ref, wcls_ref,
                cls_ref, cos_ref, vcomp_ref, vlc_ref, vlcm_ref):
    f32 = jnp.float32

    def mm(w, x):
        return jnp.dot(w, x, preferred_element_type=f32)

    xt = fct_ref[...]
    ht = _lrelu(mm(w1_ref[...], xt) + b1_ref[...])
    ht = _lrelu(mm(w2_ref[...], ht) + b2_ref[...])
    vct = _lrelu(mm(w3_ref[...], ht) + b3_ref[...])
    vloct = _lrelu(mm(wloc_ref[...], flt_ref[...]) + bloc_ref[...])

    vet = ve_ref[...].T
    n1 = jnp.sqrt(jnp.sum(vct * vct, axis=0, keepdims=True))

    best = None
    vlcmt = None
    for k in range(DICT_NUM):
        vet_k = vet[k * DICT_DIM:(k + 1) * DICT_DIM, :]
        cat_k = jnp.concatenate([vloct, vet_k], axis=0)
        vlc_ref[k, :, :] = cat_k
        dot_k = jnp.sum(vct * cat_k, axis=0, keepdims=True)
        n2_k = jnp.sqrt(jnp.sum(cat_k * cat_k, axis=0, keepdims=True))
        cos_k = dot_k / jnp.maximum(n1 * n2_k, 1e-8)
        if best is None:
            best = cos_k
            vlcmt = cat_k
        else:
            upd = cos_k > best
            best = jnp.where(upd, cos_k, best)
            vlcmt = jnp.where(upd, cat_k, vlcmt)

    cos_ref[...] = best
    vcomp_ref[...] = vct
    vlcm_ref[...] = vlcmt
    v_diff = jnp.abs(vlcmt - vct)
    cls_ref[...] = mm(wcls_ref[...], v_diff)


def _dense(feat_comp, feat_loc, v_emb, W1, b1, W2, b2, W3, b3, Wloc, bloc, Wcls):
    b = feat_comp.shape[0]
    bb = 1024
    grid = (b // bb,)
    f32 = jnp.float32

    def colt_spec(rows):
        return pl.BlockSpec((rows, bb), lambda i: (0, i))

    def full_spec(shape):
        return pl.BlockSpec(shape, lambda i: (0,) * len(shape))

    weights = [W1, b1.reshape(-1, 1), W2, b2.reshape(-1, 1),
               W3, b3.reshape(-1, 1), Wloc, bloc.reshape(-1, 1), Wcls]

    clst, cost, vcompt, vlct, vlcmt = pl.pallas_call(
        _dense_body,
        grid=grid,
        in_specs=[
            colt_spec(feat_comp.shape[1]),
            colt_spec(feat_loc.shape[1]),
            pl.BlockSpec((bb, v_emb.shape[1]), lambda i: (i, 0)),
        ] + [full_spec(w.shape) for w in weights],
        out_specs=[
            colt_spec(2),
            colt_spec(1),
            colt_spec(COMMON),
            pl.BlockSpec((DICT_NUM, COMMON, bb), lambda i: (0, 0, i)),
            colt_spec(COMMON),
        ],
        out_shape=[
            jax.ShapeDtypeStruct((2, b), f32),
            jax.ShapeDtypeStruct((1, b), f32),
            jax.ShapeDtypeStruct((COMMON, b), f32),
            jax.ShapeDtypeStruct((DICT_NUM, COMMON, b), f32),
            jax.ShapeDtypeStruct((COMMON, b), f32),
        ],
    )(feat_comp.T, feat_loc.T, v_emb, *weights)
    return (clst.T, cost.T, vcompt.T, jnp.transpose(vlct, (2, 0, 1)), vlcmt.T)


@jax.jit
def kernel(feat_comp, feat_loc, id_loc, W1, b1, W2, b2, W3, b3, emb, Wloc, bloc, Wcls):
    emb_p = _pad_table(emb)
    v_emb = _sc_gather(emb_p, id_loc.astype(jnp.int32))
    cls, cos, vcomp, vlc, vlcm = _dense(
        feat_comp, feat_loc, v_emb, W1, b1, W2, b2, W3, b3, Wloc, bloc, Wcls)
    return (cls, cos, vcomp, vlc, vlcm)


# speedup vs baseline: 7.0702x; 1.0415x over previous
import functools

import jax
import jax.numpy as jnp
from jax import lax
from jax.experimental import pallas as pl
from jax.experimental.pallas import tpu as pltpu
from jax.experimental.pallas import tpu_sc as plsc

DICT_NUM = 10
DICT_DIM = 80
COMMON = 96

_NC = 2
_NS = 16
_NW = _NC * _NS
_CHUNK = 64


def _sc_gather(emb, idx):
    vocab, d = emb.shape
    b = idx.shape[0]
    b_per_w = b // _NW
    mesh = plsc.VectorSubcoreMesh(core_axis_name="c", subcore_axis_name="s")

    @functools.partial(
        pl.kernel,
        mesh=mesh,
        out_type=jax.ShapeDtypeStruct((b, d), jnp.float32),
        scratch_types=[
            pltpu.VMEM((b_per_w,), jnp.int32),
            pltpu.VMEM((_CHUNK, d), jnp.float32),
            pltpu.VMEM((_CHUNK, d), jnp.float32),
            pltpu.SemaphoreType.DMA,
            pltpu.SemaphoreType.DMA,
            pltpu.SemaphoreType.DMA,
            pltpu.SemaphoreType.DMA,
        ],
    )
    def gather_kernel(table_hbm, idx_hbm, out_hbm, idx_v, rows_a, rows_b,
                      in_sem_a, in_sem_b, out_sem_a, out_sem_b):
        wid = lax.axis_index("s") * _NC + lax.axis_index("c")
        base = wid * b_per_w
        pltpu.sync_copy(idx_hbm.at[pl.ds(base, b_per_w)], idx_v)

        def gath(c, buf, sem):
            return pltpu.make_async_copy(
                table_hbm.at[idx_v.at[pl.ds(c, _CHUNK)]], buf, sem)

        def put(c, buf, sem):
            return pltpu.make_async_copy(
                buf, out_hbm.at[pl.ds(base + c, _CHUNK)], sem)

        gath(0, rows_a, in_sem_a).start()

        @pl.loop(0, b_per_w, step=2 * _CHUNK)
        def _(c):
            @pl.when(c > 0)
            def _():
                put(c - _CHUNK, rows_b, out_sem_b).wait()
            gath(c + _CHUNK, rows_b, in_sem_b).start()
            gath(c, rows_a, in_sem_a).wait()
            put(c, rows_a, out_sem_a).start()
            gath(c + _CHUNK, rows_b, in_sem_b).wait()
            put(c, rows_a, out_sem_a).wait()

            @pl.when(c + 2 * _CHUNK < b_per_w)
            def _():
                gath(c + 2 * _CHUNK, rows_a, in_sem_a).start()
            put(c + _CHUNK, rows_b, out_sem_b).start()

        put(b_per_w - _CHUNK, rows_b, out_sem_b).wait()

    return gather_kernel(emb, idx)


def _lrelu(x):
    return jnp.where(x >= 0, x, 0.01 * x)


def _pad_body(src_ref, dst_ref):
    dst_ref[:, :800] = src_ref[...].T
    dst_ref[:, 800:] = jnp.zeros((dst_ref.shape[0], 96), jnp.float32)


def _pad_table(emb):
    vocab = emb.shape[0]
    rb = 2048
    embt = emb.T
    return pl.pallas_call(
        _pad_body,
        grid=(pl.cdiv(vocab, rb),),
        in_specs=[pl.BlockSpec((800, rb), lambda i: (0, i))],
        out_specs=pl.BlockSpec((rb, 896), lambda i: (i, 0)),
        out_shape=jax.ShapeDtypeStruct((vocab, 896), jnp.float32),
    )(embt)


def _dense_body(fct_ref, flt_ref, ve_ref, w1_ref, b1_ref, w2_ref, b2_ref,
                w3_ref, b3_ref, wloc_ref, bloc_ref, wcls_ref,
                cls_ref, cos_ref, vcomp_ref, vlc_ref, vlcm_ref):
    f32 = jnp.float32

    def mm(w, x):
        return jnp.dot(w, x, preferred_element_type=f32)

    xt = fct_ref[...]
    ht = _lrelu(mm(w1_ref[...], xt) + b1_ref[...])
    ht = _lrelu(mm(w2_ref[...], ht) + b2_ref[...])
    vct = _lrelu(mm(w3_ref[...], ht) + b3_ref[...])
    vloct = _lrelu(mm(wloc_ref[...], flt_ref[...]) + bloc_ref[...])

    vet = ve_ref[...].T
    n1 = jnp.sqrt(jnp.sum(vct * vct, axis=0, keepdims=True))

    best = None
    vlcmt = None
    for k in range(DICT_NUM):
        vet_k = vet[k * DICT_DIM:(k + 1) * DICT_DIM, :]
        cat_k = jnp.concatenate([vloct, vet_k], axis=0)
        vlc_ref[k, :, :] = cat_k
        dot_k = jnp.sum(vct * cat_k, axis=0, keepdims=True)
        n2_k = jnp.sqrt(jnp.sum(cat_k * cat_k, axis=0, keepdims=True))
        cos_k = dot_k / jnp.maximum(n1 * n2_k, 1e-8)
        if best is None:
            best = cos_k
            vlcmt = cat_k
        else:
            upd = cos_k > best
            best = jnp.where(upd, cos_k, best)
            vlcmt = jnp.where(upd, cat_k, vlcmt)

    cos_ref[...] = best
    vcomp_ref[...] = vct
    vlcm_ref[...] = vlcmt
    v_diff = jnp.abs(vlcmt - vct)
    cls_ref[...] = mm(wcls_ref[...], v_diff)


def _dense(feat_comp, feat_loc, v_emb, W1, b1, W2, b2, W3, b3, Wloc, bloc, Wcls):
    b = feat_comp.shape[0]
    bb = 1024
    grid = (b // bb,)
    f32 = jnp.float32

    def colt_spec(rows):
        return pl.BlockSpec((rows, bb), lambda i: (0, i))

    def full_spec(shape):
        return pl.BlockSpec(shape, lambda i: (0,) * len(shape))

    weights = [W1, b1.reshape(-1, 1), W2, b2.reshape(-1, 1),
               W3, b3.reshape(-1, 1), Wloc, bloc.reshape(-1, 1), Wcls]

    clst, cost, vcompt, vlct, vlcmt = pl.pallas_call(
        _dense_body,
        grid=grid,
        in_specs=[
            colt_spec(feat_comp.shape[1]),
            colt_spec(feat_loc.shape[1]),
            pl.BlockSpec((bb, v_emb.shape[1]), lambda i: (i, 0)),
        ] + [full_spec(w.shape) for w in weights],
        out_specs=[
            colt_spec(2),
            colt_spec(1),
            colt_spec(COMMON),
            pl.BlockSpec((DICT_NUM, COMMON, bb), lambda i: (0, 0, i)),
            colt_spec(COMMON),
        ],
        out_shape=[
            jax.ShapeDtypeStruct((2, b), f32),
            jax.ShapeDtypeStruct((1, b), f32),
            jax.ShapeDtypeStruct((COMMON, b), f32),
            jax.ShapeDtypeStruct((DICT_NUM, COMMON, b), f32),
            jax.ShapeDtypeStruct((COMMON, b), f32),
        ],
    )(feat_comp.T, feat_loc.T, v_emb, *weights)
    return (clst.T, cost.T, vcompt.T, jnp.transpose(vlct, (2, 0, 1)), vlcmt.T)


@jax.jit
def kernel(feat_comp, feat_loc, id_loc, W1, b1, W2, b2, W3, b3, emb, Wloc, bloc, Wcls):
    emb_p = _pad_table(emb)
    v_emb = _sc_gather(emb_p, id_loc.astype(jnp.int32))
    cls, cos, vcomp, vlc, vlcm = _dense(
        feat_comp, feat_loc, v_emb, W1, b1, W2, b2, W3, b3, Wloc, bloc, Wcls)
    return (cls, cos, vcomp, vlc, vlcm)


# speedup vs baseline: 7.0766x; 1.0009x over previous
import functools

import jax
import jax.numpy as jnp
from jax import lax
from jax.experimental import pallas as pl
from jax.experimental.pallas import tpu as pltpu
from jax.experimental.pallas import tpu_sc as plsc

DICT_NUM = 10
DICT_DIM = 80
COMMON = 96

_NC = 2
_NS = 16
_NW = _NC * _NS
_CHUNK = 64


def _sc_gather(emb, idx):
    vocab, d = emb.shape
    b = idx.shape[0]
    b_per_w = b // _NW
    mesh = plsc.VectorSubcoreMesh(core_axis_name="c", subcore_axis_name="s")

    @functools.partial(
        pl.kernel,
        mesh=mesh,
        out_type=jax.ShapeDtypeStruct((b, d), jnp.float32),
        scratch_types=[
            pltpu.VMEM((b_per_w,), jnp.int32),
            pltpu.VMEM((_CHUNK, d), jnp.float32),
            pltpu.VMEM((_CHUNK, d), jnp.float32),
            pltpu.SemaphoreType.DMA,
            pltpu.SemaphoreType.DMA,
            pltpu.SemaphoreType.DMA,
            pltpu.SemaphoreType.DMA,
        ],
    )
    def gather_kernel(table_hbm, idx_hbm, out_hbm, idx_v, rows_a, rows_b,
                      in_sem_a, in_sem_b, out_sem_a, out_sem_b):
        wid = lax.axis_index("s") * _NC + lax.axis_index("c")
        base = wid * b_per_w
        pltpu.sync_copy(idx_hbm.at[pl.ds(base, b_per_w)], idx_v)

        def gath(c, buf, sem):
            return pltpu.make_async_copy(
                table_hbm.at[idx_v.at[pl.ds(c, _CHUNK)]], buf, sem)

        def put(c, buf, sem):
            return pltpu.make_async_copy(
                buf, out_hbm.at[pl.ds(base + c, _CHUNK)], sem)

        gath(0, rows_a, in_sem_a).start()

        @pl.loop(0, b_per_w, step=2 * _CHUNK)
        def _(c):
            @pl.when(c > 0)
            def _():
                put(c - _CHUNK, rows_b, out_sem_b).wait()
            gath(c + _CHUNK, rows_b, in_sem_b).start()
            gath(c, rows_a, in_sem_a).wait()
            put(c, rows_a, out_sem_a).start()
            gath(c + _CHUNK, rows_b, in_sem_b).wait()
            put(c, rows_a, out_sem_a).wait()

            @pl.when(c + 2 * _CHUNK < b_per_w)
            def _():
                gath(c + 2 * _CHUNK, rows_a, in_sem_a).start()
            put(c + _CHUNK, rows_b, out_sem_b).start()

        put(b_per_w - _CHUNK, rows_b, out_sem_b).wait()

    return gather_kernel(emb, idx)


def _lrelu(x):
    return jnp.where(x >= 0, x, 0.01 * x)


def _pad_body(src_ref, dst_ref):
    dst_ref[:, :800] = src_ref[...].T
    dst_ref[:, 800:] = jnp.zeros((dst_ref.shape[0], 96), jnp.float32)


def _pad_table(emb):
    vocab = emb.shape[0]
    rb = 2048
    embt = emb.T
    return pl.pallas_call(
        _pad_body,
        grid=(pl.cdiv(vocab, rb),),
        in_specs=[pl.BlockSpec((800, rb), lambda i: (0, i))],
        out_specs=pl.BlockSpec((rb, 896), lambda i: (i, 0)),
        out_shape=jax.ShapeDtypeStruct((vocab, 896), jnp.float32),
    )(embt)


def _mm(w, x):
    return jnp.dot(w, x, preferred_element_type=jnp.float32)


def _mlp_body(fct_ref, flt_ref, w1_ref, b1_ref, w2_ref, b2_ref,
              w3_ref, b3_ref, wloc_ref, bloc_ref, vcomp_ref, vloc_ref):
    xt = fct_ref[...]
    ht = _lrelu(_mm(w1_ref[...], xt) + b1_ref[...])
    ht = _lrelu(_mm(w2_ref[...], ht) + b2_ref[...])
    vcomp_ref[...] = _lrelu(_mm(w3_ref[...], ht) + b3_ref[...])
    vloc_ref[...] = _lrelu(_mm(wloc_ref[...], flt_ref[...]) + bloc_ref[...])


def _dense_body(ve_ref, vct_ref, vloct_ref, wcls_ref,
                cls_ref, cos_ref, vlc_ref, vlcm_ref):
    vct = vct_ref[...]
    vloct = vloct_ref[...]
    vet = ve_ref[...].T
    n1 = jnp.sqrt(jnp.sum(vct * vct, axis=0, keepdims=True))

    best = None
    vlcmt = None
    for k in range(DICT_NUM):
        vet_k = vet[k * DICT_DIM:(k + 1) * DICT_DIM, :]
        cat_k = jnp.concatenate([vloct, vet_k], axis=0)
        vlc_ref[k, :, :] = cat_k
        dot_k = jnp.sum(vct * cat_k, axis=0, keepdims=True)
        n2_k = jnp.sqrt(jnp.sum(cat_k * cat_k, axis=0, keepdims=True))
        cos_k = dot_k / jnp.maximum(n1 * n2_k, 1e-8)
        if best is None:
            best = cos_k
            vlcmt = cat_k
        else:
            upd = cos_k > best
            best = jnp.where(upd, cos_k, best)
            vlcmt = jnp.where(upd, cat_k, vlcmt)

    cos_ref[...] = best
    vlcm_ref[...] = vlcmt
    v_diff = jnp.abs(vlcmt - vct)
    cls_ref[...] = _mm(wcls_ref[...], v_diff)


def _mlp(feat_comp, feat_loc, W1, b1, W2, b2, W3, b3, Wloc, bloc):
    b = feat_comp.shape[0]
    bb = 2048
    f32 = jnp.float32

    def colt_spec(rows):
        return pl.BlockSpec((rows, bb), lambda i: (0, i))

    def full_spec(shape):
        return pl.BlockSpec(shape, lambda i: (0,) * len(shape))

    weights = [W1, b1.reshape(-1, 1), W2, b2.reshape(-1, 1),
               W3, b3.reshape(-1, 1), Wloc, bloc.reshape(-1, 1)]
    return pl.pallas_call(
        _mlp_body,
        grid=(b // bb,),
        in_specs=[
            colt_spec(feat_comp.shape[1]),
            colt_spec(feat_loc.shape[1]),
        ] + [full_spec(w.shape) for w in weights],
        out_specs=[colt_spec(COMMON), colt_spec(16)],
        out_shape=[
            jax.ShapeDtypeStruct((COMMON, b), f32),
            jax.ShapeDtypeStruct((16, b), f32),
        ],
    )(feat_comp.T, feat_loc.T, *weights)


def _dense(v_emb, vcompt, vloct, Wcls):
    b = v_emb.shape[0]
    bb = 1024
    grid = (b // bb,)
    f32 = jnp.float32

    def colt_spec(rows):
        return pl.BlockSpec((rows, bb), lambda i: (0, i))

    clst, cost, vlct, vlcmt = pl.pallas_call(
        _dense_body,
        grid=grid,
        in_specs=[
            pl.BlockSpec((bb, v_emb.shape[1]), lambda i: (i, 0)),
            colt_spec(COMMON),
            colt_spec(16),
            pl.BlockSpec(Wcls.shape, lambda i: (0, 0)),
        ],
        out_specs=[
            colt_spec(2),
            colt_spec(1),
            pl.BlockSpec((DICT_NUM, COMMON, bb), lambda i: (0, 0, i)),
            colt_spec(COMMON),
        ],
        out_shape=[
            jax.ShapeDtypeStruct((2, b), f32),
            jax.ShapeDtypeStruct((1, b), f32),
            jax.ShapeDtypeStruct((DICT_NUM, COMMON, b), f32),
            jax.ShapeDtypeStruct((COMMON, b), f32),
        ],
    )(v_emb, vcompt, vloct, Wcls)
    return (clst.T, cost.T, jnp.transpose(vlct, (2, 0, 1)), vlcmt.T)


@jax.jit
def kernel(feat_comp, feat_loc, id_loc, W1, b1, W2, b2, W3, b3, emb, Wloc, bloc, Wcls):
    emb_p = _pad_table(emb)
    v_emb = _sc_gather(emb_p, id_loc.astype(jnp.int32))
    vcompt, vloct = _mlp(feat_comp, feat_loc, W1, b1, W2, b2, W3, b3, Wloc, bloc)
    cls, cos, vlc, vlcm = _dense(v_emb, vcompt, vloct, Wcls)
    return (cls, cos, vcompt.T, vlc, vlcm)


# speedup vs baseline: 7.1140x; 1.0053x over previous
import functools

import jax
import jax.numpy as jnp
from jax import lax
from jax.experimental import pallas as pl
from jax.experimental.pallas import tpu as pltpu
from jax.experimental.pallas import tpu_sc as plsc

DICT_NUM = 10
DICT_DIM = 80
COMMON = 96

_NC = 2
_NS = 16
_NW = _NC * _NS
_CHUNK = 64


def _sc_gather(emb, idx):
    vocab, d = emb.shape
    b = idx.shape[0]
    b_per_w = b // _NW
    mesh = plsc.VectorSubcoreMesh(core_axis_name="c", subcore_axis_name="s")

    @functools.partial(
        pl.kernel,
        mesh=mesh,
        out_type=jax.ShapeDtypeStruct((b, d), jnp.float32),
        scratch_types=[
            pltpu.VMEM((b_per_w,), jnp.int32),
            pltpu.VMEM((_CHUNK, d), jnp.float32),
            pltpu.VMEM((_CHUNK, d), jnp.float32),
            pltpu.SemaphoreType.DMA,
            pltpu.SemaphoreType.DMA,
            pltpu.SemaphoreType.DMA,
            pltpu.SemaphoreType.DMA,
        ],
    )
    def gather_kernel(table_hbm, idx_hbm, out_hbm, idx_v, rows_a, rows_b,
                      in_sem_a, in_sem_b, out_sem_a, out_sem_b):
        wid = lax.axis_index("s") * _NC + lax.axis_index("c")
        base = wid * b_per_w
        pltpu.sync_copy(idx_hbm.at[pl.ds(base, b_per_w)], idx_v)

        def gath(c, buf, sem):
            return pltpu.make_async_copy(
                table_hbm.at[idx_v.at[pl.ds(c, _CHUNK)]], buf, sem)

        def put(c, buf, sem):
            return pltpu.make_async_copy(
                buf, out_hbm.at[pl.ds(base + c, _CHUNK)], sem)

        gath(0, rows_a, in_sem_a).start()

        @pl.loop(0, b_per_w, step=2 * _CHUNK)
        def _(c):
            @pl.when(c > 0)
            def _():
                put(c - _CHUNK, rows_b, out_sem_b).wait()
            gath(c + _CHUNK, rows_b, in_sem_b).start()
            gath(c, rows_a, in_sem_a).wait()
            put(c, rows_a, out_sem_a).start()
            gath(c + _CHUNK, rows_b, in_sem_b).wait()
            put(c, rows_a, out_sem_a).wait()

            @pl.when(c + 2 * _CHUNK < b_per_w)
            def _():
                gath(c + 2 * _CHUNK, rows_a, in_sem_a).start()
            put(c + _CHUNK, rows_b, out_sem_b).start()

        put(b_per_w - _CHUNK, rows_b, out_sem_b).wait()

    return gather_kernel(emb, idx)


def _lrelu(x):
    return jnp.where(x >= 0, x, 0.01 * x)


def _pad_body(src_ref, dst_ref):
    dst_ref[:, :800] = src_ref[...].T
    dst_ref[:, 800:] = jnp.zeros((dst_ref.shape[0], 96), jnp.float32)


def _pad_table(emb):
    vocab = emb.shape[0]
    rb = 2048
    embt = emb.T
    return pl.pallas_call(
        _pad_body,
        grid=(pl.cdiv(vocab, rb),),
        in_specs=[pl.BlockSpec((800, rb), lambda i: (0, i))],
        out_specs=pl.BlockSpec((rb, 896), lambda i: (i, 0)),
        out_shape=jax.ShapeDtypeStruct((vocab, 896), jnp.float32),
    )(embt)


def _mm(w, x):
    return jnp.dot(w, x, preferred_element_type=jnp.float32)


def _mlp_body(fct_ref, flt_ref, w1_ref, b1_ref, w2_ref, b2_ref,
              w3_ref, b3_ref, wloc_ref, bloc_ref, vcomp_ref, vloc_ref):
    xt = fct_ref[...]
    ht = _lrelu(_mm(w1_ref[...], xt) + b1_ref[...])
    ht = _lrelu(_mm(w2_ref[...], ht) + b2_ref[...])
    vcomp_ref[...] = _lrelu(_mm(w3_ref[...], ht) + b3_ref[...])
    vloc_ref[...] = _lrelu(_mm(wloc_ref[...], flt_ref[...]) + bloc_ref[...])


def _dense_body(ve_ref, vct_ref, vloct_ref, wcls_ref,
                cls_ref, cos_ref, vlc_ref, vlcm_ref):
    vct = vct_ref[...]
    vloct = vloct_ref[...]
    vet = ve_ref[...].T
    n1 = jnp.sqrt(jnp.sum(vct * vct, axis=0, keepdims=True))

    best = None
    vlcmt = None
    for k in range(DICT_NUM):
        vet_k = vet[k * DICT_DIM:(k + 1) * DICT_DIM, :]
        cat_k = jnp.concatenate([vloct, vet_k], axis=0)
        vlc_ref[k, :, :] = cat_k
        dot_k = jnp.sum(vct * cat_k, axis=0, keepdims=True)
        n2_k = jnp.sqrt(jnp.sum(cat_k * cat_k, axis=0, keepdims=True))
        cos_k = dot_k / jnp.maximum(n1 * n2_k, 1e-8)
        if best is None:
            best = cos_k
            vlcmt = cat_k
        else:
            upd = cos_k > best
            best = jnp.where(upd, cos_k, best)
            vlcmt = jnp.where(upd, cat_k, vlcmt)

    cos_ref[...] = best
    vlcm_ref[...] = vlcmt
    v_diff = jnp.abs(vlcmt - vct)
    cls_ref[...] = _mm(wcls_ref[...], v_diff)


def _mlp(feat_comp, feat_loc, W1, b1, W2, b2, W3, b3, Wloc, bloc):
    b = feat_comp.shape[0]
    bb = 2048
    f32 = jnp.float32

    def colt_spec(rows):
        return pl.BlockSpec((rows, bb), lambda i: (0, i))

    def full_spec(shape):
        return pl.BlockSpec(shape, lambda i: (0,) * len(shape))

    weights = [W1, b1.reshape(-1, 1), W2, b2.reshape(-1, 1),
               W3, b3.reshape(-1, 1), Wloc, bloc.reshape(-1, 1)]
    return pl.pallas_call(
        _mlp_body,
        grid=(b // bb,),
        in_specs=[
            colt_spec(feat_comp.shape[1]),
            colt_spec(feat_loc.shape[1]),
        ] + [full_spec(w.shape) for w in weights],
        out_specs=[colt_spec(COMMON), colt_spec(16)],
        out_shape=[
            jax.ShapeDtypeStruct((COMMON, b), f32),
            jax.ShapeDtypeStruct((16, b), f32),
        ],
    )(feat_comp.T, feat_loc.T, *weights)


def _dense(v_emb, vcompt, vloct, Wcls):
    b = v_emb.shape[0]
    bb = 2048
    grid = (b // bb,)
    f32 = jnp.float32

    def colt_spec(rows):
        return pl.BlockSpec((rows, bb), lambda i: (0, i))

    clst, cost, vlct, vlcmt = pl.pallas_call(
        _dense_body,
        grid=grid,
        in_specs=[
            pl.BlockSpec((bb, v_emb.shape[1]), lambda i: (i, 0)),
            colt_spec(COMMON),
            colt_spec(16),
            pl.BlockSpec(Wcls.shape, lambda i: (0, 0)),
        ],
        out_specs=[
            colt_spec(2),
            colt_spec(1),
            pl.BlockSpec((DICT_NUM, COMMON, bb), lambda i: (0, 0, i)),
            colt_spec(COMMON),
        ],
        out_shape=[
            jax.ShapeDtypeStruct((2, b), f32),
            jax.ShapeDtypeStruct((1, b), f32),
            jax.ShapeDtypeStruct((DICT_NUM, COMMON, b), f32),
            jax.ShapeDtypeStruct((COMMON, b), f32),
        ],
    )(v_emb, vcompt, vloct, Wcls)
    return (clst.T, cost.T, jnp.transpose(vlct, (2, 0, 1)), vlcmt.T)


@jax.jit
def kernel(feat_comp, feat_loc, id_loc, W1, b1, W2, b2, W3, b3, emb, Wloc, bloc, Wcls):
    emb_p = _pad_table(emb)
    v_emb = _sc_gather(emb_p, id_loc.astype(jnp.int32))
    vcompt, vloct = _mlp(feat_comp, feat_loc, W1, b1, W2, b2, W3, b3, Wloc, bloc)
    cls, cos, vlc, vlcm = _dense(v_emb, vcompt, vloct, Wcls)
    return (cls, cos, vcompt.T, vlc, vlcm)
